# Initial kernel scaffold; baseline (speedup 1.0000x reference)
import jax
import jax.numpy as jnp
from jax.experimental import pallas as pl


def kernel(x, pos, word_table, pos_table):
    raise NotImplementedError("write your pallas kernel here")


# speedup vs baseline: 1.2057x; 1.2057x over previous
import functools

import jax
import jax.numpy as jnp
from jax import lax
from jax.experimental import pallas as pl
from jax.experimental.pallas import tpu as pltpu
from jax.experimental.pallas import tpu_sc as plsc

_WORD_DIM = 64
_POS_DIM = 16
_OUT_DIM = _WORD_DIM + _POS_DIM
_CHUNK = 128
_LANES = 16


def _make_lookup(n_rows, pos_vocab):
    info = plsc.get_sparse_core_info()
    num_workers = info.num_cores * info.num_subcores
    per_w = n_rows // num_workers
    n_chunks = per_w // _CHUNK
    mesh = plsc.VectorSubcoreMesh(core_axis_name="c", subcore_axis_name="s")

    @functools.partial(
        pl.kernel,
        out_type=jax.ShapeDtypeStruct((n_rows, _OUT_DIM), jnp.float32),
        mesh=mesh,
        scratch_types=[
            pltpu.VMEM((per_w,), jnp.int32),
            pltpu.VMEM((per_w,), jnp.int32),
            pltpu.VMEM((_CHUNK,), jnp.int32),
            pltpu.VMEM((pos_vocab, _POS_DIM), jnp.float32),
            pltpu.VMEM((_CHUNK, 2 * _WORD_DIM), jnp.float32),
            pltpu.VMEM((_CHUNK, _OUT_DIM), jnp.float32),
            pltpu.SemaphoreType.DMA,
        ],
    )
    def lookup(x_hbm, p_hbm, wt_hbm, pt_hbm, out_hbm, xi, pi, pairs, pt_v,
               wbuf, obuf, sem):
        wid = lax.axis_index("s") * info.num_cores + lax.axis_index("c")
        base = wid * per_w
        pltpu.sync_copy(x_hbm.at[pl.ds(base, per_w)], xi)
        pltpu.sync_copy(p_hbm.at[pl.ds(base, per_w)], pi)
        pltpu.sync_copy(pt_hbm, pt_v)

        @pl.loop(0, n_chunks)
        def _(g):
            off = g * _CHUNK

            @pl.loop(0, _CHUNK // _LANES)
            def _(v):
                s = off + v * _LANES
                pairs[pl.ds(v * _LANES, _LANES)] = (
                    xi[pl.ds(s, _LANES)] >> 1)

            pltpu.async_copy(wt_hbm.at[pairs], wbuf, sem).wait()

            @pl.loop(0, _CHUNK // _LANES)
            def _(v):
                s = off + v * _LANES
                selv = (xi[pl.ds(s, _LANES)] & 1) * _WORD_DIM
                posv = pi[pl.ds(s, _LANES)]
                for j in range(_LANES):
                    row = v * _LANES + j
                    sel = selv[j]
                    for c in range(_WORD_DIM // _LANES):
                        obuf[row, pl.ds(c * _LANES, _LANES)] = (
                            wbuf[row, pl.ds(sel + c * _LANES, _LANES)])
                    obuf[row, pl.ds(_WORD_DIM, _POS_DIM)] = pt_v[posv[j]]

            pltpu.sync_copy(obuf, out_hbm.at[pl.ds(base + off, _CHUNK)])

    return lookup


def kernel(x, pos, word_table, pos_table):
    b, l = x.shape
    n_rows = b * l
    v2 = word_table.shape[0] // 2
    wt2 = word_table.reshape(v2, 2 * _WORD_DIM)
    lookup = _make_lookup(n_rows, pos_table.shape[0])
    out = lookup(x.reshape(n_rows), pos.reshape(n_rows), wt2, pos_table)
    return out.reshape(b, l, _OUT_DIM)
